# baseline (speedup 1.0000x reference)
import functools

import jax
import jax.numpy as jnp
from jax import lax
from jax.experimental import pallas as pl
from jax.experimental.pallas import tpu as pltpu
from jax.experimental.pallas import tpu_sc as plsc

N = 10000
E = 320000
D = 128
H = 16
C = 40

NC = 2
NS = 16
NW = NC * NS

BLK = 128
CHUNKS = -(-(-(-E // (NW * BLK))) // 8) * 8
EPAD = NW * BLK * CHUNKS
NROW = 10112
RPS = NROW // NS

_f32 = jnp.float32


def _sc_mesh():
    return plsc.VectorSubcoreMesh(core_axis_name="c", subcore_axis_name="s", )


def _deg_body(dst_hbm, out_hbm, dst_idx, ones_v, zbuf, acc, sem):
    cid = lax.axis_index("c")
    sid = lax.axis_index("s")
    wid = sid * NC + cid
    ebase = pl.multiple_of(wid * CHUNKS, 8)
    rbase = pl.multiple_of(sid * RPS, 8)
    obase = pl.multiple_of(cid * NROW + sid * RPS, 8)

    def fill_ones(j, _):
        ones_v[j, :] = jnp.ones((H,), _f32)
        return 0

    def fill_zero(j, _):
        zbuf[j, :] = jnp.zeros((H,), _f32)
        return 0

    lax.fori_loop(0, BLK, fill_ones, 0)
    lax.fori_loop(0, RPS, fill_zero, 0)
    pltpu.sync_copy(zbuf, acc.at[pl.ds(rbase, RPS)])
    pltpu.sync_copy(dst_hbm.at[pl.ds(ebase, CHUNKS)], dst_idx)
    plsc.subcore_barrier()

    def step(i, _):
        pltpu.async_copy(ones_v, acc.at[dst_idx.at[i]], sem, add=True)
        return 0

    lax.fori_loop(0, CHUNKS, step, 0)

    def drain(i, _):
        pltpu.make_async_copy(ones_v, acc.at[dst_idx.at[i]], sem).wait()
        return 0

    lax.fori_loop(0, CHUNKS, drain, 0)
    plsc.subcore_barrier()
    pltpu.sync_copy(acc.at[pl.ds(rbase, RPS)], zbuf)
    pltpu.sync_copy(zbuf, out_hbm.at[pl.ds(obase, RPS)])


_deg_call = pl.kernel(
    _deg_body,
    out_type=jax.ShapeDtypeStruct((NC * NROW, H), _f32),
    mesh=_sc_mesh(),
    scratch_types=[
        pltpu.VMEM((CHUNKS, BLK), jnp.int32),
        pltpu.VMEM((BLK, H), _f32),
        pltpu.VMEM((RPS, H), _f32),
        pltpu.VMEM_SHARED((NROW, H), _f32),
        pltpu.SemaphoreType.DMA,
    ],
    compiler_params=pltpu.CompilerParams(use_tc_tiling_on_sc=False),
)


GRP = 8
NG = CHUNKS // GRP


def _prop_body(y_hbm, src_hbm, dst_hbm, out_hbm,
               src_idx, dst_idx, bufs, zbuf, acc, sem_g, sem_s):
    cid = lax.axis_index("c")
    sid = lax.axis_index("s")
    wid = sid * NC + cid
    ebase = pl.multiple_of(wid * CHUNKS, 8)
    rbase = pl.multiple_of(sid * RPS, 8)
    obase = pl.multiple_of(cid * NROW + sid * RPS, 8)

    def fill_zero(j, _):
        zbuf[j, :] = jnp.zeros((H,), _f32)
        return 0

    lax.fori_loop(0, RPS, fill_zero, 0)
    pltpu.sync_copy(zbuf, acc.at[pl.ds(rbase, RPS)])
    pltpu.sync_copy(src_hbm.at[pl.ds(ebase, CHUNKS)], src_idx)
    pltpu.sync_copy(dst_hbm.at[pl.ds(ebase, CHUNKS)], dst_idx)
    plsc.subcore_barrier()

    def _buf(p, b):
        return bufs.at[pl.ds((p * GRP + b) * BLK, BLK)]

    def _wait_group(sem_ref):
        pltpu.make_async_copy(out_hbm.at[pl.ds(0, GRP * BLK)],
                              bufs.at[pl.ds(0, GRP * BLK)], sem_ref).wait()

    for b in range(GRP):
        pltpu.async_copy(y_hbm.at[src_idx.at[b]], _buf(0, b), sem_g.at[0])

    def super_group(sg, _):
        for p in range(2):
            g = sg * 2 + p
            q = 1 - p
            base = g * GRP
            _wait_group(sem_g.at[p])

            @pl.when(g >= 1)
            def _():
                _wait_group(sem_s.at[q])

            @pl.when(g + 1 < NG)
            def _():
                for b in range(GRP):
                    pltpu.async_copy(y_hbm.at[src_idx.at[base + GRP + b]],
                                     _buf(q, b), sem_g.at[q])

            for b in range(GRP):
                pltpu.async_copy(_buf(p, b), acc.at[dst_idx.at[base + b]],
                                 sem_s.at[p], add=True)
        return 0

    lax.fori_loop(0, NG // 2, super_group, 0)
    _wait_group(sem_s.at[(NG - 1) % 2])
    plsc.subcore_barrier()
    pltpu.sync_copy(acc.at[pl.ds(rbase, RPS)], zbuf)
    pltpu.sync_copy(zbuf, out_hbm.at[pl.ds(obase, RPS)])


_prop_call = pl.kernel(
    _prop_body,
    out_type=jax.ShapeDtypeStruct((NC * NROW, H), _f32),
    mesh=_sc_mesh(),
    scratch_types=[
        pltpu.VMEM((CHUNKS, BLK), jnp.int32),
        pltpu.VMEM((CHUNKS, BLK), jnp.int32),
        pltpu.VMEM((2 * GRP * BLK, H), _f32),
        pltpu.VMEM((RPS, H), _f32),
        pltpu.VMEM_SHARED((NROW, H), _f32),
        pltpu.SemaphoreType.DMA((2,)),
        pltpu.SemaphoreType.DMA((2,)),
    ],
    compiler_params=pltpu.CompilerParams(use_tc_tiling_on_sc=False),
)


def _pre_body(dd_ref, x_ref, w0_ref, dis_ref, y0_ref):
    indeg = dd_ref[0:N, :] + dd_ref[NROW:NROW + N, :]
    dis = lax.rsqrt(indeg + 1.0)
    z = jnp.dot(x_ref[...], w0_ref[...], preferred_element_type=_f32)
    dis_ref[...] = dis
    y0_ref[...] = dis * z


_pre_call = pl.pallas_call(
    _pre_body,
    out_shape=(jax.ShapeDtypeStruct((N, H), _f32),
               jax.ShapeDtypeStruct((N, H), _f32)),
)


def _layer_body(pp_ref, y_ref, dis_ref, m_ref, b_ref, w_ref,
                ynext_ref, mout_ref):
    S = pp_ref[0:N, :] + pp_ref[NROW:NROW + N, :]
    dis = dis_ref[...]
    h = jnp.maximum(dis * (S + y_ref[...]) + b_ref[...], 0.0)
    mout_ref[...] = jnp.maximum(m_ref[...], h)
    ynext_ref[...] = dis * jnp.dot(h, w_ref[...],
                                   preferred_element_type=_f32)


_layer_call = pl.pallas_call(
    _layer_body,
    out_shape=(jax.ShapeDtypeStruct((N, H), _f32),
               jax.ShapeDtypeStruct((N, H), _f32)),
)


def _final_body(pp_ref, y_ref, dis_ref, m_ref, b_ref, wfc_ref, bfc_ref,
                out_ref):
    S = pp_ref[0:N, :] + pp_ref[NROW:NROW + N, :]
    h = jnp.maximum(dis_ref[...] * (S + y_ref[...]) + b_ref[...], 0.0)
    m = jnp.maximum(m_ref[...], h)
    logits = jnp.dot(m, wfc_ref[...], preferred_element_type=_f32)
    logits = logits + bfc_ref[...]
    lmax = jnp.max(logits, axis=1, keepdims=True)
    s = logits - lmax
    out_ref[...] = s - jnp.log(jnp.sum(jnp.exp(s), axis=1, keepdims=True))


_final_call = pl.pallas_call(
    _final_body,
    out_shape=jax.ShapeDtypeStruct((N, C), _f32),
)


def kernel(x, edge_index, W0, b0, W1, b1, W2, b2, W3, b3, W4, b4, W5, b5,
           Wfc, bfc):
    src = edge_index[0].astype(jnp.int32)
    dst = edge_index[1].astype(jnp.int32)
    pad = EPAD - E
    srcr = jnp.concatenate([src, jnp.zeros((pad,), jnp.int32)])
    dstr = jnp.concatenate([dst, jnp.full((pad,), N, jnp.int32)])
    srcr = srcr.reshape(EPAD // BLK, BLK)
    dstr = dstr.reshape(EPAD // BLK, BLK)

    dd = _deg_call(dstr)
    dis, y = _pre_call(dd, x, W0)

    m = jnp.zeros((N, H), _f32)
    Ws = [W1, W2, W3, W4, W5]
    bs = [b0.reshape(1, H), b1.reshape(1, H), b2.reshape(1, H),
          b3.reshape(1, H), b4.reshape(1, H)]
    for l in range(5):
        pp = _prop_call(y, srcr, dstr)
        y, m = _layer_call(pp, y, dis, m, bs[l], Ws[l])
    pp = _prop_call(y, srcr, dstr)
    return _final_call(pp, y, dis, m, b5.reshape(1, H), Wfc,
                       bfc.reshape(1, C))


# speedup vs baseline: 1.3873x; 1.3873x over previous
import functools

import jax
import jax.numpy as jnp
from jax import lax
from jax.experimental import pallas as pl
from jax.experimental.pallas import tpu as pltpu
from jax.experimental.pallas import tpu_sc as plsc

N = 10000
E = 320000
D = 128
H = 16
C = 40

NC = 2
NS = 16
NW = NC * NS

BLK = 128
CHUNKS = -(-(-(-E // (NW * BLK))) // 8) * 8
EPAD = NW * BLK * CHUNKS
NROW = 10112
RPS = NROW // NS

_f32 = jnp.float32


def _sc_mesh():
    return plsc.VectorSubcoreMesh(core_axis_name="c", subcore_axis_name="s", )


def _deg_body(dst_hbm, out_hbm, dst_idx, ones_v, zbuf, acc, sem):
    cid = lax.axis_index("c")
    sid = lax.axis_index("s")
    wid = sid * NC + cid
    ebase = pl.multiple_of(wid * CHUNKS, 8)
    rbase = pl.multiple_of(sid * RPS, 8)
    obase = pl.multiple_of(cid * NROW + sid * RPS, 8)

    def fill_ones(j, _):
        ones_v[j, :] = jnp.ones((H,), _f32)
        return 0

    def fill_zero(j, _):
        zbuf[j, :] = jnp.zeros((H,), _f32)
        return 0

    lax.fori_loop(0, BLK, fill_ones, 0)
    lax.fori_loop(0, RPS, fill_zero, 0)
    pltpu.sync_copy(zbuf, acc.at[pl.ds(rbase, RPS)])
    pltpu.sync_copy(dst_hbm.at[pl.ds(ebase, CHUNKS)], dst_idx)
    plsc.subcore_barrier()

    def step(i, _):
        pltpu.async_copy(ones_v, acc.at[dst_idx.at[i]], sem, add=True)
        return 0

    lax.fori_loop(0, CHUNKS, step, 0)

    def drain(i, _):
        pltpu.make_async_copy(ones_v, acc.at[dst_idx.at[i]], sem).wait()
        return 0

    lax.fori_loop(0, CHUNKS, drain, 0)
    plsc.subcore_barrier()
    pltpu.sync_copy(acc.at[pl.ds(rbase, RPS)], zbuf)
    pltpu.sync_copy(zbuf, out_hbm.at[pl.ds(obase, RPS)])


_deg_call = pl.kernel(
    _deg_body,
    out_type=jax.ShapeDtypeStruct((NC * NROW, H), _f32),
    mesh=_sc_mesh(),
    scratch_types=[
        pltpu.VMEM((CHUNKS, BLK), jnp.int32),
        pltpu.VMEM((BLK, H), _f32),
        pltpu.VMEM((RPS, H), _f32),
        pltpu.VMEM_SHARED((NROW, H), _f32),
        pltpu.SemaphoreType.DMA,
    ],
    compiler_params=pltpu.CompilerParams(use_tc_tiling_on_sc=False),
)


NBUF = 4


def _prop_body(y_hbm, src_hbm, dst_hbm, out_hbm,
               src_idx, dst_idx, bufs, zbuf, acc, sem_g, sem_s):
    cid = lax.axis_index("c")
    sid = lax.axis_index("s")
    wid = sid * NC + cid
    ebase = pl.multiple_of(wid * CHUNKS, 8)
    rbase = pl.multiple_of(sid * RPS, 8)
    obase = pl.multiple_of(cid * NROW + sid * RPS, 8)

    def fill_zero(j, _):
        zbuf[j, :] = jnp.zeros((H,), _f32)
        return 0

    lax.fori_loop(0, RPS, fill_zero, 0)
    pltpu.sync_copy(zbuf, acc.at[pl.ds(rbase, RPS)])
    pltpu.sync_copy(src_hbm.at[pl.ds(ebase, CHUNKS)], src_idx)
    pltpu.sync_copy(dst_hbm.at[pl.ds(ebase, CHUNKS)], dst_idx)
    plsc.subcore_barrier()

    def _buf(b):
        return bufs.at[pl.ds(b * BLK, BLK)]

    for b in range(NBUF):
        pltpu.async_copy(y_hbm.at[src_idx.at[b]], _buf(b), sem_g.at[b])

    def group(g, _):
        for b in range(NBUF):
            i = g * NBUF + b
            pltpu.make_async_copy(y_hbm.at[src_idx.at[i]], _buf(b),
                                  sem_g.at[b]).wait()
            pltpu.sync_copy(_buf(b), acc.at[dst_idx.at[i]], add=True)

            @pl.when(i + NBUF < CHUNKS)
            def _():
                pltpu.async_copy(y_hbm.at[src_idx.at[i + NBUF]], _buf(b),
                                 sem_g.at[b])
        return 0

    lax.fori_loop(0, CHUNKS // NBUF, group, 0)
    plsc.subcore_barrier()
    pltpu.sync_copy(acc.at[pl.ds(rbase, RPS)], zbuf)
    pltpu.sync_copy(zbuf, out_hbm.at[pl.ds(obase, RPS)])


_prop_call = pl.kernel(
    _prop_body,
    out_type=jax.ShapeDtypeStruct((NC * NROW, H), _f32),
    mesh=_sc_mesh(),
    scratch_types=[
        pltpu.VMEM((CHUNKS, BLK), jnp.int32),
        pltpu.VMEM((CHUNKS, BLK), jnp.int32),
        pltpu.VMEM((NBUF * BLK, H), _f32),
        pltpu.VMEM((RPS, H), _f32),
        pltpu.VMEM_SHARED((NROW, H), _f32),
        pltpu.SemaphoreType.DMA((NBUF,)),
        pltpu.SemaphoreType.DMA((NBUF,)),
    ],
    compiler_params=pltpu.CompilerParams(use_tc_tiling_on_sc=False),
)


NP = N // 8
PROW = NROW // 8


def _pre_body(dd_ref, x_ref, w0_ref, dis_ref, y0_ref):
    indeg = dd_ref[0:NP, :] + dd_ref[PROW:PROW + NP, :]
    dis = lax.rsqrt(indeg + 1.0)
    z = jnp.dot(x_ref[...], w0_ref[...], preferred_element_type=_f32)
    dis_ref[...] = dis
    y0_ref[...] = dis * z


_pre_call = pl.pallas_call(
    _pre_body,
    out_shape=(jax.ShapeDtypeStruct((NP, 128), _f32),
               jax.ShapeDtypeStruct((NP, 128), _f32)),
)


def _layer_body(pp_ref, y_ref, dis_ref, m_ref, b_ref, w_ref,
                ynext_ref, mout_ref):
    S = pp_ref[0:NP, :] + pp_ref[PROW:PROW + NP, :]
    dis = dis_ref[...]
    h = jnp.maximum(dis * (S + y_ref[...]) + b_ref[...], 0.0)
    mout_ref[...] = jnp.maximum(m_ref[...], h)
    ynext_ref[...] = dis * jnp.dot(h, w_ref[...],
                                   preferred_element_type=_f32)


_layer_call = pl.pallas_call(
    _layer_body,
    out_shape=(jax.ShapeDtypeStruct((NP, 128), _f32),
               jax.ShapeDtypeStruct((NP, 128), _f32)),
)


def _final_body(pp_ref, y_ref, dis_ref, m_ref, b_ref, wfc_ref, bfc_ref,
                ksum_ref, kbc_ref, out_ref):
    S = pp_ref[0:NP, :] + pp_ref[PROW:PROW + NP, :]
    h = jnp.maximum(dis_ref[...] * (S + y_ref[...]) + b_ref[...], 0.0)
    m = jnp.maximum(m_ref[...], h)
    lp = jnp.dot(m, wfc_ref[...], preferred_element_type=_f32)
    lp = lp + bfc_ref[...]
    rmax = jnp.max(lp, axis=1, keepdims=True)
    e = jnp.exp(lp - rmax)
    seg = jnp.dot(e, ksum_ref[...], preferred_element_type=_f32)
    lse = jnp.dot(jnp.log(seg), kbc_ref[...],
                  preferred_element_type=_f32)
    out_ref[...] = lp - rmax - lse


_final_call = pl.pallas_call(
    _final_body,
    out_shape=jax.ShapeDtypeStruct((NP, 8 * C), _f32),
)


def kernel(x, edge_index, W0, b0, W1, b1, W2, b2, W3, b3, W4, b4, W5, b5,
           Wfc, bfc):
    src = edge_index[0].astype(jnp.int32)
    dst = edge_index[1].astype(jnp.int32)
    pad = EPAD - E
    srcr = jnp.concatenate([src, jnp.zeros((pad,), jnp.int32)])
    dstr = jnp.concatenate([dst, jnp.full((pad,), N, jnp.int32)])
    srcr = srcr.reshape(EPAD // BLK, BLK)
    dstr = dstr.reshape(EPAD // BLK, BLK)

    eye8 = jnp.eye(8, dtype=_f32)
    ksum = jnp.kron(eye8, jnp.ones((C, 1), _f32))
    kbc = jnp.kron(eye8, jnp.ones((1, C), _f32))

    dd = _deg_call(dstr)
    dis, y = _pre_call(dd.reshape(2 * PROW, 128), x.reshape(NP, 8 * D),
                       jnp.kron(eye8, W0))

    m = jnp.zeros((NP, 128), _f32)
    Ws = [W1, W2, W3, W4, W5]
    bs = [jnp.tile(b, 8).reshape(1, 128) for b in (b0, b1, b2, b3, b4)]
    for l in range(5):
        pp = _prop_call(y.reshape(N, H), srcr, dstr)
        y, m = _layer_call(pp.reshape(2 * PROW, 128), y, dis, m, bs[l],
                           jnp.kron(eye8, Ws[l]))
    pp = _prop_call(y.reshape(N, H), srcr, dstr)
    out = _final_call(pp.reshape(2 * PROW, 128), y, dis, m,
                      jnp.tile(b5, 8).reshape(1, 128),
                      jnp.kron(eye8, Wfc), jnp.tile(bfc, 8).reshape(1, 8 * C),
                      ksum, kbc)
    return out.reshape(N, C)
